# baseline (device time: 229812 ns/iter reference)
import numpy as np
import jax
import jax.numpy as jnp
from jax import lax
from jax.experimental import pallas as pl
from jax.experimental.pallas import tpu as pltpu

N_DEV = 32
CW_HOPS = N_DEV // 2
CCW_HOPS = N_DEV - 1 - CW_HOPS
N_SPLIT = 4

def _logical_coords():
    out = {}
    i = 0
    for z in range(4):
        for y in range(4):
            xs = (1, 0) if y % 2 else (0, 1)
            for x in xs:
                out[i] = (x, y, z)
                i += 1
    return out

def _ring():
    path = []
    for y in range(4):
        zs = range(3, -1, -1) if y % 2 else range(4)
        for z in zs:
            path.append((y, z))
    cycle = [(0, y, z) for (y, z) in path] + [(1, y, z) for (y, z) in reversed(path)]
    coords_to_logical = {v: k for k, v in _logical_coords().items()}
    ring = [coords_to_logical[c] for c in cycle]
    for a, b in zip(cycle, cycle[1:] + cycle[:1]):
        assert sum(abs(p - q) for p, q in zip(a, b)) == 1, (a, b)
    assert sorted(ring) == list(range(N_DEV))
    return ring

_RING = np.array(_ring(), dtype=np.int32)
_POS = np.zeros(N_DEV, dtype=np.int32)
_POS[_RING] = np.arange(N_DEV, dtype=np.int32)


def kernel(x):
    m_per, n = x.shape
    m_half = m_per // N_SPLIT

    my = lax.axis_index("i")
    ring = jnp.asarray(_RING)
    pos = jnp.asarray(_POS)[my]
    nbrs = jnp.stack([
        ring[(pos - 1) % N_DEV],
        ring[(pos + 1) % N_DEV],
    ])
    cw_slots = ring[(pos - jnp.arange(CW_HOPS)) % N_DEV]
    ccw_slots = ring[(pos + jnp.arange(CCW_HOPS)) % N_DEV]

    def body(x_ref, nbrs_ref, cw_slots_ref, ccw_slots_ref, out_ref,
             cw_send, cw_recv, ccw_send, ccw_recv):
        me = lax.axis_index("i")
        left = nbrs_ref[0]
        right = nbrs_ref[1]

        barrier_sem = pltpu.get_barrier_semaphore()
        for nbr in (left, right):
            pl.semaphore_signal(
                barrier_sem, inc=1,
                device_id=(nbr,), device_id_type=pl.DeviceIdType.MESH,
            )
        pl.semaphore_wait(barrier_sem, 2)

        out_ref[pl.ds(me * m_per, m_per), :] = x_ref[...]

        def make(slots_ref, h, s, sems_send, sems_recv, dst):
            start = slots_ref[h] * m_per + s * m_half
            return pltpu.make_async_remote_copy(
                src_ref=out_ref.at[pl.ds(start, m_half), :],
                dst_ref=out_ref.at[pl.ds(start, m_half), :],
                send_sem=sems_send.at[h, s],
                recv_sem=sems_recv.at[h, s],
                device_id=(dst,),
                device_id_type=pl.DeviceIdType.MESH,
            )

        cw = [[make(cw_slots_ref, h, s, cw_send, cw_recv, right)
               for s in range(N_SPLIT)] for h in range(CW_HOPS)]
        ccw = [[make(ccw_slots_ref, h, s, ccw_send, ccw_recv, left)
                for s in range(N_SPLIT)] for h in range(CCW_HOPS)]

        for s in range(N_SPLIT):
            cw[0][s].start()
            ccw[0][s].start()
        for h in range(CW_HOPS):
            for s in range(N_SPLIT):
                cw[h][s].wait_recv()
                if h + 1 < CW_HOPS:
                    cw[h + 1][s].start()
                if h < CCW_HOPS:
                    ccw[h][s].wait_recv()
                    if h + 1 < CCW_HOPS:
                        ccw[h + 1][s].start()

        for hops in (cw, ccw):
            for halves in hops:
                for r in halves:
                    r.wait_send()

    return pl.pallas_call(
        body,
        out_shape=jax.ShapeDtypeStruct((N_DEV * m_per, n), x.dtype),
        in_specs=[
            pl.BlockSpec(memory_space=pltpu.VMEM),
            pl.BlockSpec(memory_space=pltpu.SMEM),
            pl.BlockSpec(memory_space=pltpu.SMEM),
            pl.BlockSpec(memory_space=pltpu.SMEM),
        ],
        out_specs=pl.BlockSpec(memory_space=pltpu.VMEM),
        scratch_shapes=[
            pltpu.SemaphoreType.DMA((CW_HOPS, N_SPLIT)),
            pltpu.SemaphoreType.DMA((CW_HOPS, N_SPLIT)),
            pltpu.SemaphoreType.DMA((CCW_HOPS, N_SPLIT)),
            pltpu.SemaphoreType.DMA((CCW_HOPS, N_SPLIT)),
        ],
        compiler_params=pltpu.CompilerParams(collective_id=0),
    )(x, nbrs, cw_slots, ccw_slots)


# device time: 220296 ns/iter; 1.0432x vs baseline; 1.0432x over previous
import numpy as np
import jax
import jax.numpy as jnp
from jax import lax
from jax.experimental import pallas as pl
from jax.experimental.pallas import tpu as pltpu

N_DEV = 32
CW_HOPS = N_DEV // 2
CCW_HOPS = N_DEV - 1 - CW_HOPS
N_SPLIT = 4

def _logical_coords():
    out = {}
    i = 0
    for z in range(4):
        for y in range(4):
            xs = (1, 0) if y % 2 else (0, 1)
            for x in xs:
                out[i] = (x, y, z)
                i += 1
    return out

def _ring():
    path = []
    for y in range(4):
        zs = range(3, -1, -1) if y % 2 else range(4)
        for z in zs:
            path.append((y, z))
    cycle = [(0, y, z) for (y, z) in path] + [(1, y, z) for (y, z) in reversed(path)]
    coords_to_logical = {v: k for k, v in _logical_coords().items()}
    ring = [coords_to_logical[c] for c in cycle]
    for a, b in zip(cycle, cycle[1:] + cycle[:1]):
        assert sum(abs(p - q) for p, q in zip(a, b)) == 1, (a, b)
    assert sorted(ring) == list(range(N_DEV))
    return ring

_RING = np.array(_ring(), dtype=np.int32)
_POS = np.zeros(N_DEV, dtype=np.int32)
_POS[_RING] = np.arange(N_DEV, dtype=np.int32)


def kernel(x):
    m_per, n = x.shape
    m_half = m_per // N_SPLIT

    my = lax.axis_index("i")
    ring = jnp.asarray(_RING)
    pos = jnp.asarray(_POS)[my]
    nbrs = jnp.stack([
        ring[(pos - 1) % N_DEV],
        ring[(pos + 1) % N_DEV],
    ])
    cw_slots = ring[(pos - jnp.arange(CW_HOPS)) % N_DEV]
    ccw_slots = ring[(pos + jnp.arange(CCW_HOPS)) % N_DEV]

    def body(x_ref, nbrs_ref, cw_slots_ref, ccw_slots_ref, out_ref,
             cw_send, cw_recv, ccw_send, ccw_recv, copy_sem):
        me = lax.axis_index("i")
        left = nbrs_ref[0]
        right = nbrs_ref[1]

        barrier_sem = pltpu.get_barrier_semaphore()
        for nbr in (left, right):
            pl.semaphore_signal(
                barrier_sem, inc=1,
                device_id=(nbr,), device_id_type=pl.DeviceIdType.MESH,
            )
        pl.semaphore_wait(barrier_sem, 2)

        own = pltpu.make_async_copy(
            x_ref, out_ref.at[pl.ds(me * m_per, m_per), :], copy_sem)
        own.start()
        own.wait()

        def make(slots_ref, h, s, sems_send, sems_recv, dst):
            start = slots_ref[h] * m_per + s * m_half
            return pltpu.make_async_remote_copy(
                src_ref=out_ref.at[pl.ds(start, m_half), :],
                dst_ref=out_ref.at[pl.ds(start, m_half), :],
                send_sem=sems_send.at[h, s],
                recv_sem=sems_recv.at[h, s],
                device_id=(dst,),
                device_id_type=pl.DeviceIdType.MESH,
            )

        cw = [[make(cw_slots_ref, h, s, cw_send, cw_recv, right)
               for s in range(N_SPLIT)] for h in range(CW_HOPS)]
        ccw = [[make(ccw_slots_ref, h, s, ccw_send, ccw_recv, left)
                for s in range(N_SPLIT)] for h in range(CCW_HOPS)]

        for s in range(N_SPLIT):
            cw[0][s].start()
            ccw[0][s].start()
        for h in range(CW_HOPS):
            for s in range(N_SPLIT):
                cw[h][s].wait_recv()
                if h + 1 < CW_HOPS:
                    cw[h + 1][s].start()
                if h < CCW_HOPS:
                    ccw[h][s].wait_recv()
                    if h + 1 < CCW_HOPS:
                        ccw[h + 1][s].start()

        for hops in (cw, ccw):
            for halves in hops:
                for r in halves:
                    r.wait_send()

    return pl.pallas_call(
        body,
        out_shape=jax.ShapeDtypeStruct((N_DEV * m_per, n), x.dtype),
        in_specs=[
            pl.BlockSpec(memory_space=pltpu.VMEM),
            pl.BlockSpec(memory_space=pltpu.SMEM),
            pl.BlockSpec(memory_space=pltpu.SMEM),
            pl.BlockSpec(memory_space=pltpu.SMEM),
        ],
        out_specs=pl.BlockSpec(memory_space=pl.ANY),
        scratch_shapes=[
            pltpu.SemaphoreType.DMA((CW_HOPS, N_SPLIT)),
            pltpu.SemaphoreType.DMA((CW_HOPS, N_SPLIT)),
            pltpu.SemaphoreType.DMA((CCW_HOPS, N_SPLIT)),
            pltpu.SemaphoreType.DMA((CCW_HOPS, N_SPLIT)),
            pltpu.SemaphoreType.DMA,
        ],
        compiler_params=pltpu.CompilerParams(collective_id=0),
    )(x, nbrs, cw_slots, ccw_slots)
